# baseline (speedup 1.0000x reference)
import jax
import jax.numpy as jnp
from jax.experimental import pallas as pl
from jax.experimental.pallas import tpu as pltpu

S = 32
EPS = 1e-5


def _lnT(x):
    n = x.shape[0]
    m = jnp.sum(x, axis=0, keepdims=True) * (1.0 / n)
    d = x - m
    v = jnp.sum(d * d, axis=0, keepdims=True) * (1.0 / n)
    return d * jax.lax.rsqrt(v + EPS)


def _dot0(a, b):
    return jax.lax.dot_general(a, b, (((0,), (0,)), ((), ())),
                               preferred_element_type=jnp.float32)


def _inproj_body(x_ref, wih_ref, bias_ref, gx_ref):
    gx_ref[...] = (jnp.dot(x_ref[...], wih_ref[...],
                           preferred_element_type=jnp.float32) + bias_ref[...])


def _mega_body(gx_ref, h0_ref, c0_ref, f0_ref, whh_ref, wwr_ref, bwr_ref,
               p_ref, q_ref, wlin_ref, blin_ref, out_ref,
               h_s, c_s, F_s, n2_s, wvp_s, xp_s, qr3_s, xp2_s):
    t = pl.program_id(0)
    tlast = pl.num_programs(0) - 1
    H = c_s.shape[1]

    @pl.when(t == 0)
    def _():
        h_s[...] = h0_ref[...]
        c_s[...] = c0_ref[...]
        f0 = f0_ref[...]
        F_s[...] = f0
        ss = jnp.sum(f0 * f0, axis=2)
        n2v = jnp.sum(ss, axis=0, keepdims=True)
        n2_s[...] = jnp.broadcast_to(n2v, n2_s.shape)
        wvp_s[...] = jnp.zeros_like(wvp_s)
        qr3_s[...] = jnp.zeros_like(qr3_s)

    wv = wvp_s[...]
    xprev = xp_s[...]
    qr3 = qr3_s[...]
    xprev2 = xp2_s[...]

    h = h_s[...]
    gx = gx_ref[0]
    gl = (jnp.dot(h, whh_ref[:, 0:2 * H],
                  preferred_element_type=jnp.float32) + gx[:, 0:2 * H])
    gr = (jnp.dot(h, whh_ref[:, 2 * H:],
                  preferred_element_type=jnp.float32) + gx[:, 2 * H:])
    gi = gl[:, 0:H]
    gf = gl[:, H:2 * H]
    gg = gr[:, 0:H]
    go = gr[:, H:2 * H]
    c_new = (jax.nn.sigmoid(gf) * c_s[...]
             + jax.nn.sigmoid(gi) * jnp.tanh(gg))
    h_new = jax.nn.sigmoid(go) * jnp.tanh(c_new)
    h_s[...] = h_new
    c_s[...] = c_new
    xp_s[...] = h_new
    xp2_s[...] = xprev
    wvp_s[...] = (jnp.dot(h_new, wwr_ref[...],
                          preferred_element_type=jnp.float32)
                  + bwr_ref[...])

    bb = wv.shape[0]
    s = jnp.tanh(wv[:, 0:S])
    r = jnp.tanh(wv[:, S:2 * S])
    q0 = wv[:, 4 * S:5 * S]
    r1 = wv[:, 5 * S:6 * S]
    r2 = wv[:, 6 * S:7 * S]
    r3 = wv[:, 7 * S:8 * S]
    ttT = jnp.tanh(wv[:, 2 * S:3 * S].T)
    betaT = jax.nn.sigmoid(wv[:, 3 * S:3 * S + 1].T + 1.0)
    s2 = jnp.sum(s * s, axis=1, keepdims=True).T
    r2n = jnp.sum(r * r, axis=1, keepdims=True).T

    e1 = jnp.dot(jnp.concatenate([s, q0], axis=0), p_ref[...],
                 preferred_element_type=jnp.float32)
    e2 = jnp.dot(jnp.concatenate([r, r1, r2, r3], axis=0), q_ref[...],
                 preferred_element_type=jnp.float32)
    s_exp, q0_exp = e1[0:bb], e1[bb:2 * bb]
    r_til, r1_til, r2_til, r3_til = (e2[0:bb], e2[bb:2 * bb],
                                     e2[2 * bb:3 * bb], e2[3 * bb:4 * bb])
    sr = s_exp * r_til
    qr1 = q0_exp * r1_til

    KH = (S * S) // 2
    vT = 0.0
    h3 = 0.0
    for kk in range(2):
        sl = slice(kk * KH, (kk + 1) * KH)
        Fh = F_s[:, :, sl]
        vT = vT + jnp.sum(Fh * sr[None, :, sl], axis=2)
        h3 = h3 + jnp.sum(Fh * qr3[None, :, sl], axis=2)
    q3T = _lnT(h3)
    out_ref[0] = xprev2 + _dot0(q3T, wlin_ref[...]) + blin_ref[...]
    nvT = betaT * (ttT - vT)

    vdot = jnp.sum(vT * nvT, axis=0, keepdims=True)
    nv2 = jnp.sum(nvT * nvT, axis=0, keepdims=True)
    n2 = n2_s[0:1, :]
    n2p = n2 + (2.0 / S) * vdot + (s2 * r2n * nv2) * (1.0 / (S * S))
    scale_row = jnp.where(n2p > 1.0, jax.lax.rsqrt(n2p), 1.0)
    n2_s[...] = jnp.broadcast_to(jnp.minimum(n2p, 1.0), n2_s.shape)
    scale_col = scale_row.T

    nvs = nvT * (1.0 / S)
    h1 = 0.0
    for kk in range(2):
        sl = slice(kk * KH, (kk + 1) * KH)
        fnh = ((F_s[:, :, sl] + sr[None, :, sl] * nvs[:, :, None])
               * scale_col[None, :, :])
        F_s[:, :, sl] = fnh
        h1 = h1 + jnp.sum(fnh * qr1[None, :, sl], axis=2)
    q1T = _lnT(h1)

    q1_exp = _dot0(q1T, p_ref[...])
    qr2 = q1_exp * r2_til
    h2 = 0.0
    for kk in range(2):
        sl = slice(kk * KH, (kk + 1) * KH)
        h2 = h2 + jnp.sum(F_s[:, :, sl] * qr2[None, :, sl], axis=2)
    q2T = _lnT(h2)

    q2_exp = _dot0(q2T, p_ref[...])
    qr3_s[...] = q2_exp * r3_til


def kernel(inputs, h0, c0, F0, W_ih, W_hh, b_ih, b_hh,
           W_write, b_write, W_read, b_read, W_lin, b_lin):
    T, B, ISIZE = inputs.shape
    H = h0.shape[1]
    f32 = jnp.float32

    W_ih_t = W_ih.T
    W_hh_t = W_hh.T
    bias = (b_ih + b_hh).reshape(1, 4 * H)
    MB = 256
    x2d = inputs.reshape(T * B, ISIZE)
    gx2d = pl.pallas_call(
        _inproj_body,
        grid=(T * B // MB,),
        in_specs=[
            pl.BlockSpec((MB, ISIZE), lambda m: (m, 0)),
            pl.BlockSpec((ISIZE, 4 * H), lambda m: (0, 0)),
            pl.BlockSpec((1, 4 * H), lambda m: (0, 0)),
        ],
        out_specs=pl.BlockSpec((MB, 4 * H), lambda m: (m, 0)),
        out_shape=jax.ShapeDtypeStruct((T * B, 4 * H), f32),
        compiler_params=pltpu.CompilerParams(
            dimension_semantics=("arbitrary",),
            vmem_limit_bytes=100 * 1024 * 1024,
        ),
        name="fwm_inproj",
    )(x2d, W_ih_t, bias)

    W_wr_t = jnp.concatenate(
        [W_write, jnp.zeros((S - 1, H), f32), W_read], axis=0).T
    b_wr = jnp.concatenate(
        [b_write, jnp.zeros((S - 1,), f32), b_read]).reshape(1, 8 * S)

    F0r = F0.transpose(3, 0, 1, 2).reshape(S, B, S * S)
    ar = jnp.arange(S * S, dtype=jnp.int32)
    sidx = jnp.arange(S, dtype=jnp.int32)
    P = (ar[None, :] // S == sidx[:, None]).astype(f32)
    Q = (ar[None, :] % S == sidx[:, None]).astype(f32)
    W_lin_t = W_lin.T
    b_lin2 = b_lin.reshape(1, H)

    gxv = gx2d.reshape(T, B, 4 * H)

    out = pl.pallas_call(
        _mega_body,
        grid=(T + 2,),
        in_specs=[
            pl.BlockSpec((1, B, 4 * H), lambda t: (jnp.minimum(t, T - 1), 0, 0)),
            pl.BlockSpec((B, H), lambda t: (0, 0)),
            pl.BlockSpec((B, H), lambda t: (0, 0)),
            pl.BlockSpec((S, B, S * S), lambda t: (0, 0, 0)),
            pl.BlockSpec((H, 4 * H), lambda t: (0, 0)),
            pl.BlockSpec((H, 8 * S), lambda t: (0, 0)),
            pl.BlockSpec((1, 8 * S), lambda t: (0, 0)),
            pl.BlockSpec((S, S * S), lambda t: (0, 0)),
            pl.BlockSpec((S, S * S), lambda t: (0, 0)),
            pl.BlockSpec((S, H), lambda t: (0, 0)),
            pl.BlockSpec((1, H), lambda t: (0, 0)),
        ],
        out_specs=pl.BlockSpec((1, B, H),
                               lambda t: (jnp.maximum(t - 2, 0), 0, 0)),
        out_shape=jax.ShapeDtypeStruct((T, B, H), f32),
        scratch_shapes=[
            pltpu.VMEM((B, H), f32),
            pltpu.VMEM((B, H), f32),
            pltpu.VMEM((S, B, S * S), f32),
            pltpu.VMEM((8, B), f32),
            pltpu.VMEM((B, 8 * S), f32),
            pltpu.VMEM((B, H), f32),
            pltpu.VMEM((B, S * S), f32),
            pltpu.VMEM((B, H), f32),
        ],
        compiler_params=pltpu.CompilerParams(
            dimension_semantics=("arbitrary",),
            vmem_limit_bytes=100 * 1024 * 1024,
        ),
        name="fwm_mega",
    )(gxv, h0, c0, F0r, W_hh_t, W_wr_t, b_wr, P, Q, W_lin_t, b_lin2)
    return out


# speedup vs baseline: 1.0298x; 1.0298x over previous
import jax
import jax.numpy as jnp
from jax.experimental import pallas as pl
from jax.experimental.pallas import tpu as pltpu

S = 32
EPS = 1e-5


def _lnT(x):
    n = x.shape[0]
    m = jnp.sum(x, axis=0, keepdims=True) * (1.0 / n)
    d = x - m
    v = jnp.sum(d * d, axis=0, keepdims=True) * (1.0 / n)
    return d * jax.lax.rsqrt(v + EPS)


def _dot0(a, b):
    return jax.lax.dot_general(a, b, (((0,), (0,)), ((), ())),
                               preferred_element_type=jnp.float32)


def _inproj_body(x_ref, wih_ref, bias_ref, gx_ref):
    gx_ref[...] = (jnp.dot(x_ref[...], wih_ref[...],
                           preferred_element_type=jnp.float32) + bias_ref[...])


def _mega_body(gx_ref, h0_ref, c0_ref, f0_ref, whh_ref, wwr_ref, bwr_ref,
               p_ref, q_ref, wlin_ref, blin_ref, out_ref,
               h_s, c_s, F_s, n2_s, wvp_s, xp_s, qr3_s, xp2_s):
    t = pl.program_id(0)
    tlast = pl.num_programs(0) - 1
    H = c_s.shape[1]

    @pl.when(t == 0)
    def _():
        h_s[...] = h0_ref[...]
        c_s[...] = c0_ref[...]
        f0 = f0_ref[...]
        F_s[...] = f0
        ss = jnp.sum(f0 * f0, axis=2)
        n2v = jnp.sum(ss, axis=0, keepdims=True)
        n2_s[...] = jnp.broadcast_to(n2v, n2_s.shape)
        wvp_s[...] = jnp.zeros_like(wvp_s)
        qr3_s[...] = jnp.zeros_like(qr3_s)

    wv = wvp_s[...]
    xprev = xp_s[...]
    qr3 = qr3_s[...]
    xprev2 = xp2_s[...]

    h = h_s[...]
    gx = gx_ref[0]
    gl = (jnp.dot(h, whh_ref[:, 0:2 * H],
                  preferred_element_type=jnp.float32) + gx[:, 0:2 * H])
    gr = (jnp.dot(h, whh_ref[:, 2 * H:],
                  preferred_element_type=jnp.float32) + gx[:, 2 * H:])
    gi = gl[:, 0:H]
    gf = gl[:, H:2 * H]
    gg = gr[:, 0:H]
    go = gr[:, H:2 * H]
    c_new = (jax.nn.sigmoid(gf) * c_s[...]
             + jax.nn.sigmoid(gi) * jnp.tanh(gg))
    h_new = jax.nn.sigmoid(go) * jnp.tanh(c_new)
    h_s[...] = h_new
    c_s[...] = c_new
    xp_s[...] = h_new
    xp2_s[...] = xprev
    wvp_s[...] = (jnp.dot(h_new, wwr_ref[...],
                          preferred_element_type=jnp.float32)
                  + bwr_ref[...])

    bb = wv.shape[0]
    s = jnp.tanh(wv[:, 0:S])
    r = jnp.tanh(wv[:, S:2 * S])
    q0 = wv[:, 4 * S:5 * S]
    r1 = wv[:, 5 * S:6 * S]
    r2 = wv[:, 6 * S:7 * S]
    r3 = wv[:, 7 * S:8 * S]
    ttT = jnp.tanh(wv[:, 2 * S:3 * S].T)
    betaT = jax.nn.sigmoid(wv[:, 3 * S:3 * S + 1].T + 1.0)
    s2 = jnp.sum(s * s, axis=1, keepdims=True).T
    r2n = jnp.sum(r * r, axis=1, keepdims=True).T

    e1 = jnp.dot(jnp.concatenate([s, q0], axis=0), p_ref[...],
                 preferred_element_type=jnp.float32)
    e2 = jnp.dot(jnp.concatenate([r, r1, r2, r3], axis=0), q_ref[...],
                 preferred_element_type=jnp.float32)
    s_exp, q0_exp = e1[0:bb], e1[bb:2 * bb]
    r_til, r1_til, r2_til, r3_til = (e2[0:bb], e2[bb:2 * bb],
                                     e2[2 * bb:3 * bb], e2[3 * bb:4 * bb])
    sr = s_exp * r_til
    qr1 = q0_exp * r1_til

    F = F_s[...]
    vT = jnp.sum(F * sr[None, :, :], axis=2)
    h3 = jnp.sum(F * qr3[None, :, :], axis=2)
    q3T = _lnT(h3)
    out_ref[0] = xprev2 + _dot0(q3T, wlin_ref[...]) + blin_ref[...]
    nvT = betaT * (ttT - vT)

    vdot = jnp.sum(vT * nvT, axis=0, keepdims=True)
    nv2 = jnp.sum(nvT * nvT, axis=0, keepdims=True)
    n2 = n2_s[0:1, :]
    n2p = n2 + (2.0 / S) * vdot + (s2 * r2n * nv2) * (1.0 / (S * S))
    scale_row = jnp.where(n2p > 1.0, jax.lax.rsqrt(n2p), 1.0)
    n2_s[...] = jnp.broadcast_to(jnp.minimum(n2p, 1.0), n2_s.shape)
    scale_col = scale_row.T

    nvs = nvT * (1.0 / S)
    fn = (F + sr[None, :, :] * nvs[:, :, None]) * scale_col[None, :, :]
    F_s[...] = fn
    h1 = jnp.sum(fn * qr1[None, :, :], axis=2)
    q1T = _lnT(h1)

    q1_exp = _dot0(q1T, p_ref[...])
    h2 = jnp.sum(F_s[...] * (q1_exp * r2_til)[None, :, :], axis=2)
    q2T = _lnT(h2)

    q2_exp = _dot0(q2T, p_ref[...])
    qr3_s[...] = q2_exp * r3_til


def kernel(inputs, h0, c0, F0, W_ih, W_hh, b_ih, b_hh,
           W_write, b_write, W_read, b_read, W_lin, b_lin):
    T, B, ISIZE = inputs.shape
    H = h0.shape[1]
    f32 = jnp.float32

    W_ih_t = W_ih.T
    W_hh_t = W_hh.T
    bias = (b_ih + b_hh).reshape(1, 4 * H)
    MB = 256
    x2d = inputs.reshape(T * B, ISIZE)
    gx2d = pl.pallas_call(
        _inproj_body,
        grid=(T * B // MB,),
        in_specs=[
            pl.BlockSpec((MB, ISIZE), lambda m: (m, 0)),
            pl.BlockSpec((ISIZE, 4 * H), lambda m: (0, 0)),
            pl.BlockSpec((1, 4 * H), lambda m: (0, 0)),
        ],
        out_specs=pl.BlockSpec((MB, 4 * H), lambda m: (m, 0)),
        out_shape=jax.ShapeDtypeStruct((T * B, 4 * H), f32),
        compiler_params=pltpu.CompilerParams(
            dimension_semantics=("arbitrary",),
            vmem_limit_bytes=100 * 1024 * 1024,
        ),
        name="fwm_inproj",
    )(x2d, W_ih_t, bias)

    W_wr_t = jnp.concatenate(
        [W_write, jnp.zeros((S - 1, H), f32), W_read], axis=0).T
    b_wr = jnp.concatenate(
        [b_write, jnp.zeros((S - 1,), f32), b_read]).reshape(1, 8 * S)

    F0r = F0.transpose(3, 0, 1, 2).reshape(S, B, S * S)
    ar = jnp.arange(S * S, dtype=jnp.int32)
    sidx = jnp.arange(S, dtype=jnp.int32)
    P = (ar[None, :] // S == sidx[:, None]).astype(f32)
    Q = (ar[None, :] % S == sidx[:, None]).astype(f32)
    W_lin_t = W_lin.T
    b_lin2 = b_lin.reshape(1, H)

    gxv = gx2d.reshape(T, B, 4 * H)

    out = pl.pallas_call(
        _mega_body,
        grid=(T + 2,),
        in_specs=[
            pl.BlockSpec((1, B, 4 * H), lambda t: (jnp.minimum(t, T - 1), 0, 0)),
            pl.BlockSpec((B, H), lambda t: (0, 0)),
            pl.BlockSpec((B, H), lambda t: (0, 0)),
            pl.BlockSpec((S, B, S * S), lambda t: (0, 0, 0)),
            pl.BlockSpec((H, 4 * H), lambda t: (0, 0)),
            pl.BlockSpec((H, 8 * S), lambda t: (0, 0)),
            pl.BlockSpec((1, 8 * S), lambda t: (0, 0)),
            pl.BlockSpec((S, S * S), lambda t: (0, 0)),
            pl.BlockSpec((S, S * S), lambda t: (0, 0)),
            pl.BlockSpec((S, H), lambda t: (0, 0)),
            pl.BlockSpec((1, H), lambda t: (0, 0)),
        ],
        out_specs=pl.BlockSpec((1, B, H),
                               lambda t: (jnp.maximum(t - 2, 0), 0, 0)),
        out_shape=jax.ShapeDtypeStruct((T, B, H), f32),
        scratch_shapes=[
            pltpu.VMEM((B, H), f32),
            pltpu.VMEM((B, H), f32),
            pltpu.VMEM((S, B, S * S), f32),
            pltpu.VMEM((8, B), f32),
            pltpu.VMEM((B, 8 * S), f32),
            pltpu.VMEM((B, H), f32),
            pltpu.VMEM((B, S * S), f32),
            pltpu.VMEM((B, H), f32),
        ],
        compiler_params=pltpu.CompilerParams(
            dimension_semantics=("arbitrary",),
            vmem_limit_bytes=100 * 1024 * 1024,
        ),
        name="fwm_mega",
    )(gxv, h0, c0, F0r, W_hh_t, W_wr_t, b_wr, P, Q, W_lin_t, b_lin2)
    return out


# speedup vs baseline: 1.0322x; 1.0023x over previous
import jax
import jax.numpy as jnp
from jax.experimental import pallas as pl
from jax.experimental.pallas import tpu as pltpu

S = 32
EPS = 1e-5


def _lnT(x):
    n = x.shape[0]
    m = jnp.sum(x, axis=0, keepdims=True) * (1.0 / n)
    d = x - m
    v = jnp.sum(d * d, axis=0, keepdims=True) * (1.0 / n)
    return d * jax.lax.rsqrt(v + EPS)


def _dot0(a, b):
    return jax.lax.dot_general(a, b, (((0,), (0,)), ((), ())),
                               preferred_element_type=jnp.float32)


def _inproj_body(x_ref, wih_ref, bias_ref, gx_ref):
    gx_ref[...] = (jnp.dot(x_ref[...], wih_ref[...],
                           preferred_element_type=jnp.float32) + bias_ref[...])


def _mega_body(gx_ref, h0_ref, c0_ref, f0_ref, whh_ref, wwr_ref, bwr_ref,
               p_ref, q_ref, wlin_ref, blin_ref, out_ref,
               h_s, c_s, F_s, n2_s, wvp_s, xp_s, qr3_s, xp2_s):
    t = pl.program_id(0)
    H = c_s.shape[1]

    @pl.when(t == 0)
    def _():
        h_s[...] = h0_ref[...]
        c_s[...] = c0_ref[...]
        f0 = f0_ref[...]
        F_s[...] = f0
        ss = jnp.sum(f0 * f0, axis=2)
        n2v = jnp.sum(ss, axis=0, keepdims=True)
        n2_s[...] = jnp.broadcast_to(n2v, n2_s.shape)
        wvp_s[...] = jnp.zeros_like(wvp_s)
        qr3_s[...] = jnp.zeros_like(qr3_s)

    wv = wvp_s[...]
    xprev = xp_s[...]
    qr3 = qr3_s[...]
    xprev2 = xp2_s[...]

    h = h_s[...]
    gx = gx_ref[0]
    gl = (jnp.dot(h, whh_ref[:, 0:2 * H],
                  preferred_element_type=jnp.float32) + gx[:, 0:2 * H])
    gr = (jnp.dot(h, whh_ref[:, 2 * H:],
                  preferred_element_type=jnp.float32) + gx[:, 2 * H:])
    gi = gl[:, 0:H]
    gf = gl[:, H:2 * H]
    gg = gr[:, 0:H]
    go = gr[:, H:2 * H]
    c_new = (jax.nn.sigmoid(gf) * c_s[...]
             + jax.nn.sigmoid(gi) * jnp.tanh(gg))
    h_new = jax.nn.sigmoid(go) * jnp.tanh(c_new)
    h_s[...] = h_new
    c_s[...] = c_new
    xp_s[...] = h_new
    xp2_s[...] = xprev
    wvp_s[...] = (jnp.dot(h_new, wwr_ref[...],
                          preferred_element_type=jnp.float32)
                  + bwr_ref[...])

    bb = wv.shape[0]
    s = jnp.tanh(wv[:, 0:S])
    r = jnp.tanh(wv[:, S:2 * S])
    q0 = wv[:, 4 * S:5 * S]
    r1 = wv[:, 5 * S:6 * S]
    r2 = wv[:, 6 * S:7 * S]
    r3 = wv[:, 7 * S:8 * S]
    ttT = jnp.tanh(wv[:, 2 * S:3 * S].T)
    betaT = jax.nn.sigmoid(wv[:, 3 * S:3 * S + 1].T + 1.0)
    s2 = jnp.sum(s * s, axis=1, keepdims=True).T
    r2n = jnp.sum(r * r, axis=1, keepdims=True).T

    e1 = jnp.dot(jnp.concatenate([s, q0], axis=0), p_ref[...],
                 preferred_element_type=jnp.float32)
    e2 = jnp.dot(jnp.concatenate([r, r1, r2, r3], axis=0), q_ref[...],
                 preferred_element_type=jnp.float32)
    s_exp, q0_exp = e1[0:bb], e1[bb:2 * bb]
    r_til, r1_til, r2_til, r3_til = (e2[0:bb], e2[bb:2 * bb],
                                     e2[2 * bb:3 * bb], e2[3 * bb:4 * bb])
    sr = s_exp * r_til
    qr1 = q0_exp * r1_til

    F = F_s[...]
    vT = jnp.sum(F * sr[None, :, :], axis=2)
    h3 = jnp.sum(F * qr3[None, :, :], axis=2)
    q3T = _lnT(h3)
    out_ref[0] = xprev2 + _dot0(q3T, wlin_ref[...]) + blin_ref[...]
    nvT = betaT * (ttT - vT)

    vdot = jnp.sum(vT * nvT, axis=0, keepdims=True)
    nv2 = jnp.sum(nvT * nvT, axis=0, keepdims=True)
    n2 = n2_s[0:1, :]
    n2p = n2 + (2.0 / S) * vdot + (s2 * r2n * nv2) * (1.0 / (S * S))
    scale_row = jnp.where(n2p > 1.0, jax.lax.rsqrt(n2p), 1.0)
    n2_s[...] = jnp.broadcast_to(jnp.minimum(n2p, 1.0), n2_s.shape)
    scale_col = scale_row.T

    nvs = nvT * (1.0 / S)
    fn = (F + sr[None, :, :] * nvs[:, :, None]) * scale_col[None, :, :]
    F_s[...] = fn
    h1 = jnp.sum(fn * qr1[None, :, :], axis=2)
    q1T = _lnT(h1)

    q1_exp = _dot0(q1T, p_ref[...])
    h2 = jnp.sum(F_s[...] * (q1_exp * r2_til)[None, :, :], axis=2)
    q2T = _lnT(h2)

    q2_exp = _dot0(q2T, p_ref[...])
    qr3_s[...] = q2_exp * r3_til


def kernel(inputs, h0, c0, F0, W_ih, W_hh, b_ih, b_hh,
           W_write, b_write, W_read, b_read, W_lin, b_lin):
    T, B, ISIZE = inputs.shape
    H = h0.shape[1]
    f32 = jnp.float32

    W_ih_t = W_ih.T
    W_hh_t = W_hh.T
    bias = (b_ih + b_hh).reshape(1, 4 * H)
    MB = 256
    x2d = inputs.reshape(T * B, ISIZE)
    gx2d = pl.pallas_call(
        _inproj_body,
        grid=(T * B // MB,),
        in_specs=[
            pl.BlockSpec((MB, ISIZE), lambda m: (m, 0)),
            pl.BlockSpec((ISIZE, 4 * H), lambda m: (0, 0)),
            pl.BlockSpec((1, 4 * H), lambda m: (0, 0)),
        ],
        out_specs=pl.BlockSpec((MB, 4 * H), lambda m: (m, 0)),
        out_shape=jax.ShapeDtypeStruct((T * B, 4 * H), f32),
        compiler_params=pltpu.CompilerParams(
            dimension_semantics=("arbitrary",),
            vmem_limit_bytes=100 * 1024 * 1024,
        ),
        name="fwm_inproj",
    )(x2d, W_ih_t, bias)

    W_wr_t = jnp.concatenate(
        [W_write, jnp.zeros((S - 1, H), f32), W_read], axis=0).T
    b_wr = jnp.concatenate(
        [b_write, jnp.zeros((S - 1,), f32), b_read]).reshape(1, 8 * S)

    F0r = F0.transpose(3, 0, 1, 2).reshape(S, B, S * S)
    ar = jnp.arange(S * S, dtype=jnp.int32)
    sidx = jnp.arange(S, dtype=jnp.int32)
    P = (ar[None, :] // S == sidx[:, None]).astype(f32)
    Q = (ar[None, :] % S == sidx[:, None]).astype(f32)
    W_lin_t = W_lin.T
    b_lin2 = b_lin.reshape(1, H)

    gxv = gx2d.reshape(T, B, 4 * H)

    out = pl.pallas_call(
        _mega_body,
        grid=(T + 2,),
        in_specs=[
            pl.BlockSpec((1, B, 4 * H), lambda t: (jnp.minimum(t, T - 1), 0, 0)),
            pl.BlockSpec((B, H), lambda t: (0, 0)),
            pl.BlockSpec((B, H), lambda t: (0, 0)),
            pl.BlockSpec((S, B, S * S), lambda t: (0, 0, 0)),
            pl.BlockSpec((H, 4 * H), lambda t: (0, 0)),
            pl.BlockSpec((H, 8 * S), lambda t: (0, 0)),
            pl.BlockSpec((1, 8 * S), lambda t: (0, 0)),
            pl.BlockSpec((S, S * S), lambda t: (0, 0)),
            pl.BlockSpec((S, S * S), lambda t: (0, 0)),
            pl.BlockSpec((S, H), lambda t: (0, 0)),
            pl.BlockSpec((1, H), lambda t: (0, 0)),
        ],
        out_specs=pl.BlockSpec((1, B, H),
                               lambda t: (jnp.maximum(t - 2, 0), 0, 0)),
        out_shape=jax.ShapeDtypeStruct((T, B, H), f32),
        scratch_shapes=[
            pltpu.VMEM((B, H), f32),
            pltpu.VMEM((B, H), f32),
            pltpu.VMEM((S, B, S * S), f32),
            pltpu.VMEM((8, B), f32),
            pltpu.VMEM((B, 8 * S), f32),
            pltpu.VMEM((B, H), f32),
            pltpu.VMEM((B, S * S), f32),
            pltpu.VMEM((B, H), f32),
        ],
        compiler_params=pltpu.CompilerParams(
            dimension_semantics=("arbitrary",),
            vmem_limit_bytes=100 * 1024 * 1024,
        ),
        name="fwm_mega",
    )(gxv, h0, c0, F0r, W_hh_t, W_wr_t, b_wr, P, Q, W_lin_t, b_lin2)
    return out
